# Initial kernel scaffold; baseline (speedup 1.0000x reference)
import jax
import jax.numpy as jnp
from jax.experimental import pallas as pl


def kernel(input_ids, tf_type, idf_type, word_table, pos_table, type_table, tf_table, idf_table, gamma, beta):
    raise NotImplementedError("write your pallas kernel here")


# speedup vs baseline: 2.4613x; 2.4613x over previous
import functools

import jax
import jax.numpy as jnp
from jax import lax
from jax.experimental import pallas as pl
from jax.experimental.pallas import tpu as pltpu
from jax.experimental.pallas import tpu_sc as plsc

HID = 1024
EPS = 1e-12


def _sc_gather(word_table, flat_ids):
    n = flat_ids.shape[0]
    info = plsc.get_sparse_core_info()
    nw = info.num_cores * info.num_subcores
    per_w = n // nw
    chunk = 64
    n_chunks = per_w // chunk
    mesh = plsc.VectorSubcoreMesh(core_axis_name="c", subcore_axis_name="s")

    @functools.partial(
        pl.kernel,
        mesh=mesh,
        out_type=jax.ShapeDtypeStruct((n, HID), jnp.float32),
        scratch_types=[
            pltpu.VMEM((chunk,), jnp.int32),
            pltpu.VMEM((chunk, HID), jnp.float32),
            pltpu.SemaphoreType.DMA,
        ],
    )
    def gather_kernel(ids_hbm, table_hbm, out_hbm, idx_v, rows_v, sem):
        wid = lax.axis_index("s") * info.num_cores + lax.axis_index("c")
        base = wid * per_w

        def body(c, carry):
            tok = base + c * chunk
            pltpu.sync_copy(ids_hbm.at[pl.ds(tok, chunk)], idx_v)
            pltpu.async_copy(table_hbm.at[idx_v], rows_v, sem).wait()
            pltpu.sync_copy(rows_v, out_hbm.at[pl.ds(tok, chunk)])
            return carry

        lax.fori_loop(0, n_chunks, body, 0)

    return gather_kernel(flat_ids, word_table)


def _tc_add_ln(gathered, pos_s, const_row, dtf, didf, tf_f, idf_f, gamma, beta):
    n = gathered.shape[0]
    blk = 256
    s_blocks = pos_s.shape[0] // blk

    def body(g_ref, p_ref, c_ref, dtf_ref, didf_ref, tf_ref, idf_ref,
             gam_ref, bet_ref, o_ref):
        x = g_ref[...] + p_ref[...] + c_ref[...]
        x = x + tf_ref[...] * dtf_ref[...]
        x = x + idf_ref[...] * didf_ref[...]
        mu = jnp.mean(x, axis=-1, keepdims=True)
        xc = x - mu
        var = jnp.mean(xc * xc, axis=-1, keepdims=True)
        y = xc * lax.rsqrt(var + EPS)
        o_ref[...] = y * gam_ref[...] + bet_ref[...]

    row_spec = pl.BlockSpec((1, HID), lambda i: (0, 0))
    return pl.pallas_call(
        body,
        grid=(n // blk,),
        in_specs=[
            pl.BlockSpec((blk, HID), lambda i: (i, 0)),
            pl.BlockSpec((blk, HID), lambda i: (i % s_blocks, 0)),
            row_spec,
            row_spec,
            row_spec,
            pl.BlockSpec((blk, 1), lambda i: (i, 0)),
            pl.BlockSpec((blk, 1), lambda i: (i, 0)),
            row_spec,
            row_spec,
        ],
        out_specs=pl.BlockSpec((blk, HID), lambda i: (i, 0)),
        out_shape=jax.ShapeDtypeStruct((n, HID), jnp.float32),
    )(gathered, pos_s, const_row, dtf, didf, tf_f, idf_f, gamma, beta)


def kernel(input_ids, tf_type, idf_type, word_table, pos_table, type_table,
           tf_table, idf_table, gamma, beta):
    b, s = input_ids.shape
    flat_ids = input_ids.reshape(-1).astype(jnp.int32)

    gathered = _sc_gather(word_table, flat_ids)

    pos_s = pos_table[:s]
    const_row = (type_table[0] + tf_table[0] + idf_table[0])[None, :]
    dtf = (tf_table[1] - tf_table[0])[None, :]
    didf = (idf_table[1] - idf_table[0])[None, :]
    tf_f = tf_type.reshape(-1, 1).astype(jnp.float32)
    idf_f = idf_type.reshape(-1, 1).astype(jnp.float32)

    out = _tc_add_ln(gathered, pos_s, const_row, dtf, didf, tf_f, idf_f,
                     gamma[None, :], beta[None, :])
    return out.reshape(b, s, HID)


# speedup vs baseline: 2.9032x; 1.1796x over previous
import functools

import jax
import jax.numpy as jnp
from jax import lax
from jax.experimental import pallas as pl
from jax.experimental.pallas import tpu as pltpu
from jax.experimental.pallas import tpu_sc as plsc

HID = 1024
EPS = 1e-12


def _sc_gather(word_table, flat_ids):
    n = flat_ids.shape[0]
    info = plsc.get_sparse_core_info()
    nw = info.num_cores * info.num_subcores
    per_w = n // nw
    chunk = 64
    n_chunks = per_w // chunk
    mesh = plsc.VectorSubcoreMesh(core_axis_name="c", subcore_axis_name="s")

    @functools.partial(
        pl.kernel,
        mesh=mesh,
        out_type=jax.ShapeDtypeStruct((n, HID), jnp.float32),
        scratch_types=[
            pltpu.VMEM((chunk,), jnp.int32),
            pltpu.VMEM((chunk, HID), jnp.float32),
            pltpu.SemaphoreType.DMA,
        ],
    )
    def gather_kernel(ids_hbm, table_hbm, out_hbm, idx_v, rows_v, sem):
        wid = lax.axis_index("s") * info.num_cores + lax.axis_index("c")
        base = wid * per_w

        def body(c, carry):
            tok = base + c * chunk
            pltpu.sync_copy(ids_hbm.at[pl.ds(tok, chunk)], idx_v)
            pltpu.async_copy(table_hbm.at[idx_v], rows_v, sem).wait()
            pltpu.sync_copy(rows_v, out_hbm.at[pl.ds(tok, chunk)])
            return carry

        lax.fori_loop(0, n_chunks, body, 0)

    return gather_kernel(flat_ids, word_table)


def _tc_add_ln(gathered, pos_s, const_row, dtf, didf, tf_f, idf_f, gamma, beta):
    b, s, _ = gathered.shape
    blk = 256
    s_blocks = s // blk

    def body(g_ref, p_ref, c_ref, dtf_ref, didf_ref, tf_ref, idf_ref,
             gam_ref, bet_ref, o_ref):
        x = g_ref[...] + p_ref[...][None] + c_ref[...][None]
        x = x + tf_ref[...] * dtf_ref[...][None]
        x = x + idf_ref[...] * didf_ref[...][None]
        mu = jnp.mean(x, axis=-1, keepdims=True)
        xc = x - mu
        var = jnp.mean(xc * xc, axis=-1, keepdims=True)
        y = xc * lax.rsqrt(var + EPS)
        o_ref[...] = y * gam_ref[...][None] + bet_ref[...][None]

    row_spec = pl.BlockSpec((1, HID), lambda i: (0, 0))
    return pl.pallas_call(
        body,
        grid=(s_blocks,),
        in_specs=[
            pl.BlockSpec((b, blk, HID), lambda i: (0, i, 0)),
            pl.BlockSpec((blk, HID), lambda i: (i, 0)),
            row_spec,
            row_spec,
            row_spec,
            pl.BlockSpec((b, blk, 1), lambda i: (0, i, 0)),
            pl.BlockSpec((b, blk, 1), lambda i: (0, i, 0)),
            row_spec,
            row_spec,
        ],
        out_specs=pl.BlockSpec((b, blk, HID), lambda i: (0, i, 0)),
        out_shape=jax.ShapeDtypeStruct((b, s, HID), jnp.float32),
    )(gathered, pos_s, const_row, dtf, didf, tf_f, idf_f, gamma, beta)


def kernel(input_ids, tf_type, idf_type, word_table, pos_table, type_table,
           tf_table, idf_table, gamma, beta):
    b, s = input_ids.shape
    flat_ids = input_ids.reshape(-1).astype(jnp.int32)

    gathered = _sc_gather(word_table, flat_ids).reshape(b, s, HID)

    pos_s = pos_table[:s]
    const_row = (type_table[0] + tf_table[0] + idf_table[0])[None, :]
    dtf = (tf_table[1] - tf_table[0])[None, :]
    didf = (idf_table[1] - idf_table[0])[None, :]
    tf_f = tf_type.reshape(b, s, 1).astype(jnp.float32)
    idf_f = idf_type.reshape(b, s, 1).astype(jnp.float32)

    return _tc_add_ln(gathered, pos_s, const_row, dtf, didf, tf_f, idf_f,
                      gamma[None, :], beta[None, :])
